# baseline (speedup 1.0000x reference)
import functools

import jax
import jax.numpy as jnp
from jax import lax
from jax.experimental import pallas as pl
from jax.experimental.pallas import tpu as pltpu
from jax.experimental.pallas import tpu_sc as plsc

IN_CH = 128
OUT_CH = 128
RADIUS = 0.015
NUM_IN = 8192
NUM_OUT = 8192
E_MAX = 2 * NUM_OUT
OB = 256
FB = 1024
EB = 512


def _radius_kernel(op2_ref, ipt_ref, ip2_ref, opos_ref, sm_ref, rc_ref):
    ob = opos_ref[...]
    mm = jnp.dot(ob, ipt_ref[...])
    d2 = (op2_ref[...] + ip2_ref[...]) - 2.0 * mm
    hit = d2 <= jnp.float32(RADIUS * RADIUS)
    srcid = lax.broadcasted_iota(jnp.int32, (OB, NUM_IN), 1) + 1
    sm_ref[...] = jnp.where(hit, srcid, 0)
    rc_ref[...] = jnp.sum(hit.astype(jnp.int32), axis=1, keepdims=True)


def _radius_search(out_positions, inp_positions):
    op2 = jnp.sum(out_positions ** 2, axis=1)[:, None]
    ip2 = jnp.sum(inp_positions ** 2, axis=1)[None, :]
    ipt = inp_positions.T
    grid = NUM_OUT // OB
    srcmask, rowcnt = pl.pallas_call(
        _radius_kernel,
        grid=(grid,),
        in_specs=[
            pl.BlockSpec((OB, 1), lambda i: (i, 0)),
            pl.BlockSpec((3, NUM_IN), lambda i: (0, 0)),
            pl.BlockSpec((1, NUM_IN), lambda i: (0, 0)),
            pl.BlockSpec((OB, 3), lambda i: (i, 0)),
        ],
        out_specs=[
            pl.BlockSpec((OB, NUM_IN), lambda i: (i, 0)),
            pl.BlockSpec((OB, 1), lambda i: (i, 0)),
        ],
        out_shape=[
            jax.ShapeDtypeStruct((NUM_OUT, NUM_IN), jnp.int32),
            jax.ShapeDtypeStruct((NUM_OUT, 1), jnp.int32),
        ],
    )(op2, ipt, ip2, out_positions)
    return srcmask, rowcnt.reshape(NUM_OUT)


CB = 256


def _compact_kernel(rc_ref, offs_ref, sm_ref, dst_ref, src_ref):
    i = pl.program_id(0)

    @pl.when(i == 0)
    def _init():
        dst_ref[...] = jnp.zeros_like(dst_ref)
        src_ref[...] = jnp.zeros_like(src_ref)

    iota128 = lax.broadcasted_iota(jnp.int32, (1, 128), 1)
    iota8 = lax.broadcasted_iota(jnp.int32, (8, 128), 0)

    def row_body(rl, carry):
        row = i * CB + rl
        cnt = rc_ref[0, row]
        slot0 = offs_ref[0, row]

        @pl.when((cnt > 0) & (slot0 < E_MAX))
        def _scan_row():
            def chunk_body(ch, slot):
                blk = sm_ref[pl.ds(pl.multiple_of((rl // 8) * 8, 8), 8),
                             pl.ds(ch * 128, 128)]
                c = jnp.max(jnp.where(iota8 == rl % 8, blk, 0),
                            axis=0, keepdims=True)

                def cond(st):
                    return jnp.max(st[0]) > 0

                def body(st):
                    c2, s2 = st
                    lane = jnp.min(jnp.where(c2 > 0, iota128, NUM_IN))

                    @pl.when(s2 < E_MAX)
                    def _st():
                        dst_ref[pl.ds(s2, 1), :] = jnp.full(
                            (1, 1), row, jnp.int32)
                        src_ref[pl.ds(s2, 1), :] = jnp.full(
                            (1, 1), ch * 128 + lane, jnp.int32)

                    return jnp.where(iota128 == lane, 0, c2), s2 + 1

                _, slot = lax.while_loop(cond, body, (c, slot))
                return slot

            lax.fori_loop(0, NUM_IN // 128, chunk_body, slot0)

        return carry

    lax.fori_loop(0, CB, row_body, jnp.int32(0))


def _compact(srcmask, rowcnt, offs):
    grid = NUM_OUT // CB
    dst, src = pl.pallas_call(
        _compact_kernel,
        grid=(grid,),
        in_specs=[
            pl.BlockSpec(memory_space=pltpu.SMEM),
            pl.BlockSpec(memory_space=pltpu.SMEM),
            pl.BlockSpec((CB, NUM_IN), lambda i: (i, 0)),
        ],
        out_specs=[
            pl.BlockSpec((E_MAX, 1), lambda i: (0, 0)),
            pl.BlockSpec((E_MAX, 1), lambda i: (0, 0)),
        ],
        out_shape=[
            jax.ShapeDtypeStruct((E_MAX, 1), jnp.int32),
            jax.ShapeDtypeStruct((E_MAX, 1), jnp.int32),
        ],
    )(rowcnt[None, :], offs[None, :], srcmask)
    return dst.reshape(E_MAX), src.reshape(E_MAX)


def _make_sc_gather():
    info = plsc.get_sparse_core_info()
    nc, ns = info.num_cores, info.num_subcores
    nw = nc * ns
    bpw = E_MAX // nw
    mesh = plsc.VectorSubcoreMesh(core_axis_name="c", subcore_axis_name="s")

    @functools.partial(
        pl.kernel,
        mesh=mesh,
        out_type=(
            jax.ShapeDtypeStruct((E_MAX, IN_CH), jnp.float32),
            jax.ShapeDtypeStruct((E_MAX, 128), jnp.float32),
            jax.ShapeDtypeStruct((E_MAX, 128), jnp.float32),
        ),
        scratch_types=[
            pltpu.VMEM((bpw,), jnp.int32),
            pltpu.VMEM((bpw,), jnp.int32),
            pltpu.VMEM((bpw, 128), jnp.float32),
            pltpu.SemaphoreType.DMA,
        ],
    )
    def gather_k(x_hbm, po_hbm, pi_hbm, src_hbm, dst_hbm,
                 xsrc_out, oat_out, iat_out,
                 src_v, dst_v, rows_v, sem):
        wid = lax.axis_index("s") * nc + lax.axis_index("c")
        base = wid * bpw
        pltpu.sync_copy(src_hbm.at[pl.ds(base, bpw)], src_v)
        pltpu.sync_copy(dst_hbm.at[pl.ds(base, bpw)], dst_v)
        pltpu.async_copy(x_hbm.at[src_v], rows_v, sem).wait()
        pltpu.sync_copy(rows_v, xsrc_out.at[pl.ds(base, bpw)])
        pltpu.async_copy(po_hbm.at[dst_v], rows_v, sem).wait()
        pltpu.sync_copy(rows_v, oat_out.at[pl.ds(base, bpw)])
        pltpu.async_copy(pi_hbm.at[src_v], rows_v, sem).wait()
        pltpu.sync_copy(rows_v, iat_out.at[pl.ds(base, bpw)])

    return gather_k


def _mlp_kernel(ne_ref, oat_ref, iat_ref, x_ref,
                w1o_ref, w1i_ref, b1_ref, w2_ref, b2_ref, w3_ref, b3_ref,
                w4_ref, b4r_ref, out_ref, cnt_ref):
    base = pl.program_id(0) * EB
    ne = ne_ref[0]

    @pl.when(base >= ne)
    def _skip():
        out_ref[...] = jnp.zeros_like(out_ref)
        cnt_ref[...] = jnp.zeros_like(cnt_ref)

    @pl.when(base < ne)
    def _compute():
        h = oat_ref[...] @ w1o_ref[...] + iat_ref[...] @ w1i_ref[...]
        h = jnp.maximum(h + b1_ref[...], 0.0)
        h = jnp.maximum(h @ w2_ref[...] + b2_ref[...], 0.0)
        h = jnp.maximum(h @ w3_ref[...] + b3_ref[...], 0.0)
        xb = x_ref[...]
        acc = xb @ b4r_ref[...]
        for k in range(100):
            acc = acc + h[:, k:k + 1] * (xb @ w4_ref[k])
        eidx = base + lax.broadcasted_iota(jnp.int32, (EB, 1), 0)
        vmask = (eidx < ne).astype(jnp.float32)
        out_ref[...] = acc * vmask
        cnt_ref[...] = jnp.broadcast_to(vmask, (EB, OUT_CH))


def _edge_messages(n_edges, oat, iat, xsrc, w1o, w1i, b1, W2, b2, W3, b3,
                   w4r, b4r):
    grid = E_MAX // EB
    full = lambda a: pl.BlockSpec(a.shape, lambda i: tuple(0 for _ in a.shape))
    return pl.pallas_call(
        _mlp_kernel,
        grid=(grid,),
        in_specs=[
            pl.BlockSpec(memory_space=pltpu.SMEM),
            pl.BlockSpec((EB, 128), lambda i: (i, 0)),
            pl.BlockSpec((EB, 128), lambda i: (i, 0)),
            pl.BlockSpec((EB, IN_CH), lambda i: (i, 0)),
            full(w1o), full(w1i), full(b1), full(W2), full(b2),
            full(W3), full(b3), full(w4r), full(b4r),
        ],
        out_specs=[
            pl.BlockSpec((EB, OUT_CH), lambda i: (i, 0)),
            pl.BlockSpec((EB, OUT_CH), lambda i: (i, 0)),
        ],
        out_shape=[
            jax.ShapeDtypeStruct((E_MAX, OUT_CH), jnp.float32),
            jax.ShapeDtypeStruct((E_MAX, OUT_CH), jnp.float32),
        ],
    )(n_edges, oat, iat, xsrc, w1o, w1i, b1, W2, b2, W3, b3, w4r, b4r)


def _make_sc_scatter():
    info = plsc.get_sparse_core_info()
    nc, ns = info.num_cores, info.num_subcores
    half = NUM_OUT // nc
    rowblk = half // 2
    chunk = 512
    nchunk = E_MAX // (ns * chunk)
    rows_ps = rowblk // ns
    mesh = plsc.VectorSubcoreMesh(core_axis_name="c", subcore_axis_name="s")

    @functools.partial(
        pl.kernel,
        mesh=mesh,
        out_type=jax.ShapeDtypeStruct((NUM_OUT, OUT_CH), jnp.float32),
        scratch_types=[
            pltpu.VMEM((chunk,), jnp.int32),
            pltpu.VMEM((chunk, OUT_CH), jnp.float32),
            pltpu.VMEM((rows_ps, OUT_CH), jnp.float32),
            pltpu.VMEM_SHARED((rowblk + 16, OUT_CH), jnp.float32),
            pltpu.SemaphoreType.DMA,
        ],
    )
    def scatter_k(rows_hbm, dst_hbm, zeros_hbm, out_hbm,
                  idx_v, buf_v, drain_v, acc_sh, sem):
        c = lax.axis_index("c")
        s = lax.axis_index("s")
        rbase = s * rows_ps
        for rp in range(2):
            row_lo = c * half + rp * rowblk
            pltpu.sync_copy(zeros_hbm, drain_v)
            pltpu.sync_copy(drain_v, acc_sh.at[pl.ds(rbase, rows_ps)])
            plsc.subcore_barrier()
            for ch in range(nchunk):
                base = ch * (ns * chunk) + s * chunk
                pltpu.sync_copy(dst_hbm.at[pl.ds(base, chunk)], idx_v)
                for j in range(chunk // 16):
                    v = idx_v[pl.ds(j * 16, 16)] - row_lo
                    keep = (v >= 0) & (v < rowblk)
                    idx_v[pl.ds(j * 16, 16)] = jnp.where(keep, v, rowblk)
                pltpu.sync_copy(rows_hbm.at[pl.ds(base, chunk)], buf_v)
                pltpu.sync_copy(buf_v, acc_sh.at[idx_v], add=True)
            plsc.subcore_barrier()
            pltpu.sync_copy(acc_sh.at[pl.ds(rbase, rows_ps)], drain_v)
            pltpu.sync_copy(drain_v, out_hbm.at[pl.ds(row_lo + rbase, rows_ps)])
            plsc.subcore_barrier()

    return scatter_k


def _finalize_kernel(s_ref, c_ref, bias_ref, out_ref):
    cnt = c_ref[:, 0:1]
    out_ref[...] = s_ref[...] / jnp.maximum(cnt, 1.0) + bias_ref[...]


def _finalize(s_msg, s_cnt, conv_bias):
    grid = NUM_OUT // FB
    return pl.pallas_call(
        _finalize_kernel,
        grid=(grid,),
        in_specs=[
            pl.BlockSpec((FB, OUT_CH), lambda i: (i, 0)),
            pl.BlockSpec((FB, OUT_CH), lambda i: (i, 0)),
            pl.BlockSpec((1, OUT_CH), lambda i: (0, 0)),
        ],
        out_specs=pl.BlockSpec((FB, OUT_CH), lambda i: (i, 0)),
        out_shape=jax.ShapeDtypeStruct((NUM_OUT, OUT_CH), jnp.float32),
    )(s_msg, s_cnt, conv_bias[None, :])


def kernel(x, inp_positions, out_positions, W1, b1, W2, b2, W3, b3, W4, b4,
           lin_w, conv_bias):
    x2 = x.reshape(NUM_IN, IN_CH)

    srcmask, rowcnt = _radius_search(out_positions, inp_positions)
    n_edges = jnp.sum(rowcnt)
    offs = (jnp.cumsum(rowcnt) - rowcnt).astype(jnp.int32)
    dst, src = _compact(srcmask, rowcnt, offs)

    po128 = jnp.pad(out_positions, ((0, 0), (0, 125)))
    pi128 = jnp.pad(inp_positions, ((0, 0), (0, 125)))
    xsrc, oat, iat = _make_sc_gather()(x2, po128, pi128, src, dst)

    w1o = jnp.pad(W1[:3], ((0, 125), (0, 0)))
    w1i = jnp.pad(W1[3:], ((0, 125), (0, 0)))
    w4r = W4.reshape(100, IN_CH, OUT_CH)
    b4r = b4.reshape(IN_CH, OUT_CH)
    msg, cntrow = _edge_messages(n_edges.reshape(1), oat, iat, xsrc,
                                 w1o, w1i, b1[None, :], W2, b2[None, :], W3,
                                 b3[None, :], w4r, b4r)

    zeros = jnp.zeros((NUM_OUT // 2 // 2 // 16, OUT_CH), jnp.float32)
    scatter = _make_sc_scatter()
    s_msg = scatter(msg, dst, zeros)
    s_cnt = scatter(cntrow, dst, zeros)

    out = _finalize(s_msg, s_cnt, conv_bias)
    return out.reshape(1, NUM_OUT, OUT_CH)


# speedup vs baseline: 48.0621x; 48.0621x over previous
import functools

import jax
import jax.numpy as jnp
from jax import lax
from jax.experimental import pallas as pl
from jax.experimental.pallas import tpu as pltpu
from jax.experimental.pallas import tpu_sc as plsc

IN_CH = 128
OUT_CH = 128
RADIUS = 0.015
NUM_IN = 8192
NUM_OUT = 8192
E_MAX = 2 * NUM_OUT
OB = 256
FB = 1024
EB = 512


NCH = NUM_IN // 128


def _radius_kernel(op2_ref, ipt_ref, ip2_ref, opos_ref, b64_ref,
                   sm_ref, cc_ref):
    ob = opos_ref[...]
    mm = jnp.dot(ob, ipt_ref[...])
    d2 = (op2_ref[...] + ip2_ref[...]) - 2.0 * mm
    hit = d2 <= jnp.float32(RADIUS * RADIUS)
    srcid = lax.broadcasted_iota(jnp.int32, (OB, NUM_IN), 1) + 1
    sm_ref[...] = jnp.where(hit, srcid, 0)
    cc_ref[...] = jnp.dot(hit.astype(jnp.float32),
                          b64_ref[...]).astype(jnp.int32)


def _radius_search(out_positions, inp_positions):
    op2 = jnp.sum(out_positions ** 2, axis=1)[:, None]
    ip2 = jnp.sum(inp_positions ** 2, axis=1)[None, :]
    ipt = inp_positions.T
    lane = jnp.arange(NUM_IN)
    b64 = (lane[:, None] // 128 == jnp.arange(NCH)[None, :]).astype(jnp.float32)
    grid = NUM_OUT // OB
    srcmask, cnt128 = pl.pallas_call(
        _radius_kernel,
        grid=(grid,),
        in_specs=[
            pl.BlockSpec((OB, 1), lambda i: (i, 0)),
            pl.BlockSpec((3, NUM_IN), lambda i: (0, 0)),
            pl.BlockSpec((1, NUM_IN), lambda i: (0, 0)),
            pl.BlockSpec((OB, 3), lambda i: (i, 0)),
            pl.BlockSpec((NUM_IN, NCH), lambda i: (0, 0)),
        ],
        out_specs=[
            pl.BlockSpec((OB, NUM_IN), lambda i: (i, 0)),
            pl.BlockSpec((OB, NCH), lambda i: (i, 0)),
        ],
        out_shape=[
            jax.ShapeDtypeStruct((NUM_OUT, NUM_IN), jnp.int32),
            jax.ShapeDtypeStruct((NUM_OUT, NCH), jnp.int32),
        ],
    )(op2, ipt, ip2, out_positions, b64)
    return srcmask, cnt128


def _compact(srcmask, cnt128):
    ccf = cnt128.reshape(-1)
    n_chunks = jnp.sum((ccf > 0).astype(jnp.int32))
    (pair_idx,) = jnp.nonzero(ccf, size=E_MAX, fill_value=0)
    pair_idx = pair_idx.astype(jnp.int32)
    cand = srcmask.reshape(-1, 128)[pair_idx]
    live = jnp.arange(E_MAX) < n_chunks
    cand = jnp.where(live[:, None], cand, 0)
    pidx, lane = jnp.nonzero(cand, size=E_MAX, fill_value=0)
    pr = pair_idx[pidx]
    dst = (pr // NCH).astype(jnp.int32)
    src = ((pr % NCH) * 128 + lane).astype(jnp.int32)
    return dst, src


def _make_sc_gather():
    info = plsc.get_sparse_core_info()
    nc, ns = info.num_cores, info.num_subcores
    nw = nc * ns
    bpw = E_MAX // nw
    mesh = plsc.VectorSubcoreMesh(core_axis_name="c", subcore_axis_name="s")

    @functools.partial(
        pl.kernel,
        mesh=mesh,
        out_type=(
            jax.ShapeDtypeStruct((E_MAX, IN_CH), jnp.float32),
            jax.ShapeDtypeStruct((E_MAX, 128), jnp.float32),
            jax.ShapeDtypeStruct((E_MAX, 128), jnp.float32),
        ),
        scratch_types=[
            pltpu.VMEM((bpw,), jnp.int32),
            pltpu.VMEM((bpw,), jnp.int32),
            pltpu.VMEM((bpw, 128), jnp.float32),
            pltpu.SemaphoreType.DMA,
        ],
    )
    def gather_k(x_hbm, po_hbm, pi_hbm, src_hbm, dst_hbm,
                 xsrc_out, oat_out, iat_out,
                 src_v, dst_v, rows_v, sem):
        wid = lax.axis_index("s") * nc + lax.axis_index("c")
        base = wid * bpw
        pltpu.sync_copy(src_hbm.at[pl.ds(base, bpw)], src_v)
        pltpu.sync_copy(dst_hbm.at[pl.ds(base, bpw)], dst_v)
        pltpu.async_copy(x_hbm.at[src_v], rows_v, sem).wait()
        pltpu.sync_copy(rows_v, xsrc_out.at[pl.ds(base, bpw)])
        pltpu.async_copy(po_hbm.at[dst_v], rows_v, sem).wait()
        pltpu.sync_copy(rows_v, oat_out.at[pl.ds(base, bpw)])
        pltpu.async_copy(pi_hbm.at[src_v], rows_v, sem).wait()
        pltpu.sync_copy(rows_v, iat_out.at[pl.ds(base, bpw)])

    return gather_k


def _mlp_kernel(ne_ref, oat_ref, iat_ref, x_ref,
                w1o_ref, w1i_ref, b1_ref, w2_ref, b2_ref, w3_ref, b3_ref,
                w4_ref, b4r_ref, out_ref, cnt_ref):
    base = pl.program_id(0) * EB
    ne = ne_ref[0]

    @pl.when(base >= ne)
    def _skip():
        out_ref[...] = jnp.zeros_like(out_ref)
        cnt_ref[...] = jnp.zeros_like(cnt_ref)

    @pl.when(base < ne)
    def _compute():
        h = oat_ref[...] @ w1o_ref[...] + iat_ref[...] @ w1i_ref[...]
        h = jnp.maximum(h + b1_ref[...], 0.0)
        h = jnp.maximum(h @ w2_ref[...] + b2_ref[...], 0.0)
        h = jnp.maximum(h @ w3_ref[...] + b3_ref[...], 0.0)
        xb = x_ref[...]
        acc = xb @ b4r_ref[...]
        for k in range(100):
            acc = acc + h[:, k:k + 1] * (xb @ w4_ref[k])
        eidx = base + lax.broadcasted_iota(jnp.int32, (EB, 1), 0)
        vmask = (eidx < ne).astype(jnp.float32)
        out_ref[...] = acc * vmask
        cnt_ref[...] = jnp.broadcast_to(vmask, (EB, OUT_CH))


def _edge_messages(n_edges, oat, iat, xsrc, w1o, w1i, b1, W2, b2, W3, b3,
                   w4r, b4r):
    grid = E_MAX // EB
    full = lambda a: pl.BlockSpec(a.shape, lambda i: tuple(0 for _ in a.shape))
    return pl.pallas_call(
        _mlp_kernel,
        grid=(grid,),
        in_specs=[
            pl.BlockSpec(memory_space=pltpu.SMEM),
            pl.BlockSpec((EB, 128), lambda i: (i, 0)),
            pl.BlockSpec((EB, 128), lambda i: (i, 0)),
            pl.BlockSpec((EB, IN_CH), lambda i: (i, 0)),
            full(w1o), full(w1i), full(b1), full(W2), full(b2),
            full(W3), full(b3), full(w4r), full(b4r),
        ],
        out_specs=[
            pl.BlockSpec((EB, OUT_CH), lambda i: (i, 0)),
            pl.BlockSpec((EB, OUT_CH), lambda i: (i, 0)),
        ],
        out_shape=[
            jax.ShapeDtypeStruct((E_MAX, OUT_CH), jnp.float32),
            jax.ShapeDtypeStruct((E_MAX, OUT_CH), jnp.float32),
        ],
    )(n_edges, oat, iat, xsrc, w1o, w1i, b1, W2, b2, W3, b3, w4r, b4r)


def _make_sc_scatter():
    info = plsc.get_sparse_core_info()
    nc, ns = info.num_cores, info.num_subcores
    half = NUM_OUT // nc
    rowblk = half // 2
    chunk = 512
    nchunk = E_MAX // (ns * chunk)
    rows_ps = rowblk // ns
    mesh = plsc.VectorSubcoreMesh(core_axis_name="c", subcore_axis_name="s")

    @functools.partial(
        pl.kernel,
        mesh=mesh,
        out_type=jax.ShapeDtypeStruct((NUM_OUT, OUT_CH), jnp.float32),
        scratch_types=[
            pltpu.VMEM((chunk,), jnp.int32),
            pltpu.VMEM((chunk, OUT_CH), jnp.float32),
            pltpu.VMEM((rows_ps, OUT_CH), jnp.float32),
            pltpu.VMEM_SHARED((rowblk + 16, OUT_CH), jnp.float32),
            pltpu.SemaphoreType.DMA,
        ],
    )
    def scatter_k(rows_hbm, dst_hbm, zeros_hbm, out_hbm,
                  idx_v, buf_v, drain_v, acc_sh, sem):
        c = lax.axis_index("c")
        s = lax.axis_index("s")
        rbase = s * rows_ps
        for rp in range(2):
            row_lo = c * half + rp * rowblk
            pltpu.sync_copy(zeros_hbm, drain_v)
            pltpu.sync_copy(drain_v, acc_sh.at[pl.ds(rbase, rows_ps)])
            plsc.subcore_barrier()
            for ch in range(nchunk):
                base = ch * (ns * chunk) + s * chunk
                pltpu.sync_copy(dst_hbm.at[pl.ds(base, chunk)], idx_v)
                for j in range(chunk // 16):
                    v = idx_v[pl.ds(j * 16, 16)] - row_lo
                    keep = (v >= 0) & (v < rowblk)
                    idx_v[pl.ds(j * 16, 16)] = jnp.where(keep, v, rowblk)
                pltpu.sync_copy(rows_hbm.at[pl.ds(base, chunk)], buf_v)
                pltpu.sync_copy(buf_v, acc_sh.at[idx_v], add=True)
            plsc.subcore_barrier()
            pltpu.sync_copy(acc_sh.at[pl.ds(rbase, rows_ps)], drain_v)
            pltpu.sync_copy(drain_v, out_hbm.at[pl.ds(row_lo + rbase, rows_ps)])
            plsc.subcore_barrier()

    return scatter_k


def _finalize_kernel(s_ref, c_ref, bias_ref, out_ref):
    cnt = c_ref[:, 0:1]
    out_ref[...] = s_ref[...] / jnp.maximum(cnt, 1.0) + bias_ref[...]


def _finalize(s_msg, s_cnt, conv_bias):
    grid = NUM_OUT // FB
    return pl.pallas_call(
        _finalize_kernel,
        grid=(grid,),
        in_specs=[
            pl.BlockSpec((FB, OUT_CH), lambda i: (i, 0)),
            pl.BlockSpec((FB, OUT_CH), lambda i: (i, 0)),
            pl.BlockSpec((1, OUT_CH), lambda i: (0, 0)),
        ],
        out_specs=pl.BlockSpec((FB, OUT_CH), lambda i: (i, 0)),
        out_shape=jax.ShapeDtypeStruct((NUM_OUT, OUT_CH), jnp.float32),
    )(s_msg, s_cnt, conv_bias[None, :])


def kernel(x, inp_positions, out_positions, W1, b1, W2, b2, W3, b3, W4, b4,
           lin_w, conv_bias):
    x2 = x.reshape(NUM_IN, IN_CH)

    srcmask, cnt128 = _radius_search(out_positions, inp_positions)
    n_edges = jnp.sum(cnt128)
    dst, src = _compact(srcmask, cnt128)

    po128 = jnp.pad(out_positions, ((0, 0), (0, 125)))
    pi128 = jnp.pad(inp_positions, ((0, 0), (0, 125)))
    xsrc, oat, iat = _make_sc_gather()(x2, po128, pi128, src, dst)

    w1o = jnp.pad(W1[:3], ((0, 125), (0, 0)))
    w1i = jnp.pad(W1[3:], ((0, 125), (0, 0)))
    w4r = W4.reshape(100, IN_CH, OUT_CH)
    b4r = b4.reshape(IN_CH, OUT_CH)
    msg, cntrow = _edge_messages(n_edges.reshape(1), oat, iat, xsrc,
                                 w1o, w1i, b1[None, :], W2, b2[None, :], W3,
                                 b3[None, :], w4r, b4r)

    zeros = jnp.zeros((NUM_OUT // 2 // 2 // 16, OUT_CH), jnp.float32)
    scatter = _make_sc_scatter()
    s_msg = scatter(msg, dst, zeros)
    s_cnt = scatter(cntrow, dst, zeros)

    out = _finalize(s_msg, s_cnt, conv_bias)
    return out.reshape(1, NUM_OUT, OUT_CH)


# speedup vs baseline: 51.0826x; 1.0628x over previous
import functools

import jax
import jax.numpy as jnp
from jax import lax
from jax.experimental import pallas as pl
from jax.experimental.pallas import tpu as pltpu
from jax.experimental.pallas import tpu_sc as plsc

IN_CH = 128
OUT_CH = 128
RADIUS = 0.015
NUM_IN = 8192
NUM_OUT = 8192
E_MAX = 2 * NUM_OUT
OB = 256
FB = 1024
EB = 512


NCH = NUM_IN // 128


def _radius_kernel(op2_ref, ipt_ref, ip2_ref, opos_ref, b64_ref,
                   sm_ref, cc_ref):
    ob = opos_ref[...]
    mm = jnp.dot(ob, ipt_ref[...])
    d2 = (op2_ref[...] + ip2_ref[...]) - 2.0 * mm
    hit = d2 <= jnp.float32(RADIUS * RADIUS)
    sm_ref[...] = hit.astype(jnp.int8)
    cc_ref[...] = jnp.dot(hit.astype(jnp.float32),
                          b64_ref[...]).astype(jnp.int32)


def _radius_search(out_positions, inp_positions):
    op2 = jnp.sum(out_positions ** 2, axis=1)[:, None]
    ip2 = jnp.sum(inp_positions ** 2, axis=1)[None, :]
    ipt = inp_positions.T
    lane = jnp.arange(NUM_IN)
    b64 = (lane[:, None] // 128 == jnp.arange(NCH)[None, :]).astype(jnp.float32)
    grid = NUM_OUT // OB
    srcmask, cnt128 = pl.pallas_call(
        _radius_kernel,
        grid=(grid,),
        in_specs=[
            pl.BlockSpec((OB, 1), lambda i: (i, 0)),
            pl.BlockSpec((3, NUM_IN), lambda i: (0, 0)),
            pl.BlockSpec((1, NUM_IN), lambda i: (0, 0)),
            pl.BlockSpec((OB, 3), lambda i: (i, 0)),
            pl.BlockSpec((NUM_IN, NCH), lambda i: (0, 0)),
        ],
        out_specs=[
            pl.BlockSpec((OB, NUM_IN), lambda i: (i, 0)),
            pl.BlockSpec((OB, NCH), lambda i: (i, 0)),
        ],
        out_shape=[
            jax.ShapeDtypeStruct((NUM_OUT, NUM_IN), jnp.int8),
            jax.ShapeDtypeStruct((NUM_OUT, NCH), jnp.int32),
        ],
    )(op2, ipt, ip2, out_positions, b64)
    return srcmask, cnt128


def _compact(srcmask, cnt128):
    ccf = cnt128.reshape(-1)
    n_chunks = jnp.sum((ccf > 0).astype(jnp.int32))
    (pair_idx,) = jnp.nonzero(ccf, size=E_MAX, fill_value=0)
    pair_idx = pair_idx.astype(jnp.int32)
    cand = srcmask.reshape(-1, 128)[pair_idx]
    live = jnp.arange(E_MAX) < n_chunks
    cand = jnp.where(live[:, None], cand, 0)
    pidx, lane = jnp.nonzero(cand, size=E_MAX, fill_value=0)
    pr = pair_idx[pidx]
    dst = (pr // NCH).astype(jnp.int32)
    src = ((pr % NCH) * 128 + lane).astype(jnp.int32)
    return dst, src


def _make_sc_gather():
    info = plsc.get_sparse_core_info()
    nc, ns = info.num_cores, info.num_subcores
    nw = nc * ns
    bpw = E_MAX // nw
    mesh = plsc.VectorSubcoreMesh(core_axis_name="c", subcore_axis_name="s")

    @functools.partial(
        pl.kernel,
        mesh=mesh,
        out_type=(
            jax.ShapeDtypeStruct((E_MAX, IN_CH), jnp.float32),
            jax.ShapeDtypeStruct((E_MAX, 128), jnp.float32),
            jax.ShapeDtypeStruct((E_MAX, 128), jnp.float32),
        ),
        scratch_types=[
            pltpu.VMEM((bpw,), jnp.int32),
            pltpu.VMEM((bpw,), jnp.int32),
            pltpu.VMEM((bpw, 128), jnp.float32),
            pltpu.SemaphoreType.DMA,
        ],
    )
    def gather_k(x_hbm, po_hbm, pi_hbm, src_hbm, dst_hbm,
                 xsrc_out, oat_out, iat_out,
                 src_v, dst_v, rows_v, sem):
        wid = lax.axis_index("s") * nc + lax.axis_index("c")
        base = wid * bpw
        pltpu.sync_copy(src_hbm.at[pl.ds(base, bpw)], src_v)
        pltpu.sync_copy(dst_hbm.at[pl.ds(base, bpw)], dst_v)
        pltpu.async_copy(x_hbm.at[src_v], rows_v, sem).wait()
        pltpu.sync_copy(rows_v, xsrc_out.at[pl.ds(base, bpw)])
        pltpu.async_copy(po_hbm.at[dst_v], rows_v, sem).wait()
        pltpu.sync_copy(rows_v, oat_out.at[pl.ds(base, bpw)])
        pltpu.async_copy(pi_hbm.at[src_v], rows_v, sem).wait()
        pltpu.sync_copy(rows_v, iat_out.at[pl.ds(base, bpw)])

    return gather_k


def _mlp_kernel(ne_ref, oat_ref, iat_ref, x_ref,
                w1o_ref, w1i_ref, b1_ref, w2_ref, b2_ref, w3_ref, b3_ref,
                w4_ref, b4r_ref, out_ref, cnt_ref):
    base = pl.program_id(0) * EB
    ne = ne_ref[0]

    @pl.when(base >= ne)
    def _skip():
        out_ref[...] = jnp.zeros_like(out_ref)
        cnt_ref[...] = jnp.zeros_like(cnt_ref)

    @pl.when(base < ne)
    def _compute():
        h = oat_ref[...] @ w1o_ref[...] + iat_ref[...] @ w1i_ref[...]
        h = jnp.maximum(h + b1_ref[...], 0.0)
        h = jnp.maximum(h @ w2_ref[...] + b2_ref[...], 0.0)
        h = jnp.maximum(h @ w3_ref[...] + b3_ref[...], 0.0)
        xb = x_ref[...]
        acc = xb @ b4r_ref[...]
        for k in range(100):
            acc = acc + h[:, k:k + 1] * (xb @ w4_ref[k])
        eidx = base + lax.broadcasted_iota(jnp.int32, (EB, 1), 0)
        vmask = (eidx < ne).astype(jnp.float32)
        out_ref[...] = acc * vmask
        cnt_ref[...] = jnp.broadcast_to(vmask, (EB, OUT_CH))


def _edge_messages(n_edges, oat, iat, xsrc, w1o, w1i, b1, W2, b2, W3, b3,
                   w4r, b4r):
    grid = E_MAX // EB
    full = lambda a: pl.BlockSpec(a.shape, lambda i: tuple(0 for _ in a.shape))
    return pl.pallas_call(
        _mlp_kernel,
        grid=(grid,),
        in_specs=[
            pl.BlockSpec(memory_space=pltpu.SMEM),
            pl.BlockSpec((EB, 128), lambda i: (i, 0)),
            pl.BlockSpec((EB, 128), lambda i: (i, 0)),
            pl.BlockSpec((EB, IN_CH), lambda i: (i, 0)),
            full(w1o), full(w1i), full(b1), full(W2), full(b2),
            full(W3), full(b3), full(w4r), full(b4r),
        ],
        out_specs=[
            pl.BlockSpec((EB, OUT_CH), lambda i: (i, 0)),
            pl.BlockSpec((EB, OUT_CH), lambda i: (i, 0)),
        ],
        out_shape=[
            jax.ShapeDtypeStruct((E_MAX, OUT_CH), jnp.float32),
            jax.ShapeDtypeStruct((E_MAX, OUT_CH), jnp.float32),
        ],
    )(n_edges, oat, iat, xsrc, w1o, w1i, b1, W2, b2, W3, b3, w4r, b4r)


def _make_sc_scatter():
    info = plsc.get_sparse_core_info()
    nc, ns = info.num_cores, info.num_subcores
    half = NUM_OUT // nc
    rowblk = half // 2
    chunk = 512
    nchunk = E_MAX // (ns * chunk)
    rows_ps = rowblk // ns
    mesh = plsc.VectorSubcoreMesh(core_axis_name="c", subcore_axis_name="s")

    @functools.partial(
        pl.kernel,
        mesh=mesh,
        out_type=jax.ShapeDtypeStruct((NUM_OUT, OUT_CH), jnp.float32),
        scratch_types=[
            pltpu.VMEM((chunk,), jnp.int32),
            pltpu.VMEM((chunk, OUT_CH), jnp.float32),
            pltpu.VMEM((rows_ps, OUT_CH), jnp.float32),
            pltpu.VMEM_SHARED((rowblk + 16, OUT_CH), jnp.float32),
            pltpu.SemaphoreType.DMA,
        ],
    )
    def scatter_k(rows_hbm, dst_hbm, zeros_hbm, out_hbm,
                  idx_v, buf_v, drain_v, acc_sh, sem):
        c = lax.axis_index("c")
        s = lax.axis_index("s")
        rbase = s * rows_ps
        for rp in range(2):
            row_lo = c * half + rp * rowblk
            pltpu.sync_copy(zeros_hbm, drain_v)
            pltpu.sync_copy(drain_v, acc_sh.at[pl.ds(rbase, rows_ps)])
            plsc.subcore_barrier()
            for ch in range(nchunk):
                base = ch * (ns * chunk) + s * chunk
                pltpu.sync_copy(dst_hbm.at[pl.ds(base, chunk)], idx_v)
                for j in range(chunk // 16):
                    v = idx_v[pl.ds(j * 16, 16)] - row_lo
                    keep = (v >= 0) & (v < rowblk)
                    idx_v[pl.ds(j * 16, 16)] = jnp.where(keep, v, rowblk)
                pltpu.sync_copy(rows_hbm.at[pl.ds(base, chunk)], buf_v)
                pltpu.sync_copy(buf_v, acc_sh.at[idx_v], add=True)
            plsc.subcore_barrier()
            pltpu.sync_copy(acc_sh.at[pl.ds(rbase, rows_ps)], drain_v)
            pltpu.sync_copy(drain_v, out_hbm.at[pl.ds(row_lo + rbase, rows_ps)])
            plsc.subcore_barrier()

    return scatter_k


def _finalize_kernel(s_ref, c_ref, bias_ref, out_ref):
    cnt = c_ref[:, 0:1]
    out_ref[...] = s_ref[...] / jnp.maximum(cnt, 1.0) + bias_ref[...]


def _finalize(s_msg, s_cnt, conv_bias):
    grid = NUM_OUT // FB
    return pl.pallas_call(
        _finalize_kernel,
        grid=(grid,),
        in_specs=[
            pl.BlockSpec((FB, OUT_CH), lambda i: (i, 0)),
            pl.BlockSpec((FB, OUT_CH), lambda i: (i, 0)),
            pl.BlockSpec((1, OUT_CH), lambda i: (0, 0)),
        ],
        out_specs=pl.BlockSpec((FB, OUT_CH), lambda i: (i, 0)),
        out_shape=jax.ShapeDtypeStruct((NUM_OUT, OUT_CH), jnp.float32),
    )(s_msg, s_cnt, conv_bias[None, :])


def kernel(x, inp_positions, out_positions, W1, b1, W2, b2, W3, b3, W4, b4,
           lin_w, conv_bias):
    x2 = x.reshape(NUM_IN, IN_CH)

    srcmask, cnt128 = _radius_search(out_positions, inp_positions)
    n_edges = jnp.sum(cnt128)
    dst, src = _compact(srcmask, cnt128)

    po128 = jnp.pad(out_positions, ((0, 0), (0, 125)))
    pi128 = jnp.pad(inp_positions, ((0, 0), (0, 125)))
    xsrc, oat, iat = _make_sc_gather()(x2, po128, pi128, src, dst)

    w1o = jnp.pad(W1[:3], ((0, 125), (0, 0)))
    w1i = jnp.pad(W1[3:], ((0, 125), (0, 0)))
    w4r = W4.reshape(100, IN_CH, OUT_CH)
    b4r = b4.reshape(IN_CH, OUT_CH)
    msg, cntrow = _edge_messages(n_edges.reshape(1), oat, iat, xsrc,
                                 w1o, w1i, b1[None, :], W2, b2[None, :], W3,
                                 b3[None, :], w4r, b4r)

    zeros = jnp.zeros((NUM_OUT // 2 // 2 // 16, OUT_CH), jnp.float32)
    scatter = _make_sc_scatter()
    s_msg = scatter(msg, dst, zeros)
    s_cnt = scatter(cntrow, dst, zeros)

    out = _finalize(s_msg, s_cnt, conv_bias)
    return out.reshape(1, NUM_OUT, OUT_CH)
